# baseline (speedup 1.0000x reference)
import functools

import jax
import jax.numpy as jnp
from jax import lax
from jax.experimental import pallas as pl
from jax.experimental.pallas import tpu as pltpu
from jax.experimental.pallas import tpu_sc as plsc

NUM_WORKERS = 32
CHUNK = 800
NB = 4
K = 2
BB = 2048


@functools.lru_cache(maxsize=None)
def _build_gather(n_total, vocab, dim):
  per_w = n_total // NUM_WORKERS
  assert per_w * NUM_WORKERS == n_total
  n = per_w // CHUNK
  assert n * CHUNK == per_w and n % NB == 0 and n // NB >= 2
  mesh = plsc.VectorSubcoreMesh(core_axis_name="c", subcore_axis_name="s")

  @functools.partial(
      pl.kernel,
      mesh=mesh,
      out_type=jax.ShapeDtypeStruct((n_total, dim), jnp.float32),
      compiler_params=pltpu.CompilerParams(use_tc_tiling_on_sc=False),
      scratch_types=(
          [pltpu.VMEM((NB, CHUNK), jnp.int32),
           pltpu.VMEM((NB, CHUNK, dim), jnp.float32)]
          + [pltpu.SemaphoreType.DMA] * (3 * NB)
      ),
  )
  def emb(idx_hbm, table_hbm, out_hbm, idx_v, rows_v, *sems):
    lsem = sems[0:NB]
    gsem = sems[NB:2 * NB]
    ssem = sems[2 * NB:3 * NB]
    wid = lax.axis_index("s") * 2 + lax.axis_index("c")
    base = wid * per_w

    def idx_load(g, b):
      return pltpu.make_async_copy(
          idx_hbm.at[pl.ds(base + g * CHUNK, CHUNK)], idx_v.at[b], lsem[b])

    def gath(b):
      return pltpu.make_async_copy(
          table_hbm.at[idx_v.at[b]], rows_v.at[b], gsem[b])

    def store(g, b):
      return pltpu.make_async_copy(
          rows_v.at[b], out_hbm.at[pl.ds(base + g * CHUNK, CHUNK)], ssem[b])

    for b in range(NB):
      idx_load(b, b).start()
    for j in range(K):
      idx_load(j, j).wait()
      gath(j).start()

    def step(g, b, do_idx_load, do_store_wait):
      gath(b).wait()
      store(g, b).start()
      if do_idx_load:
        idx_load(g + NB, b).start()
      b2 = (b + K) % NB
      if do_store_wait:
        store(g + K - NB, b2).wait()
      idx_load(g + K, b2).wait()
      gath(b2).start()

    for b in range(NB):
      step(b, b, True, b + K >= NB)

    def outer(go, carry):
      for b in range(NB):
        step(go * NB + b, b, True, True)
      return carry

    lax.fori_loop(1, n // NB - 1, outer, 0)

    for b in range(NB):
      g = n - NB + b
      gath(b).wait()
      store(g, b).start()
      if g + K < n:
        b2 = (b + K) % NB
        store(g + K - NB, b2).wait()
        idx_load(g + K, b2).wait()
        gath(b2).start()
    for b in range(NB):
      store(n - NB + b, b).wait()

  return emb


@functools.lru_cache(maxsize=None)
def _build_transpose(hist, batch, dim):
  packed_rows = BB * dim // 128
  lanes_per_row = 128 // dim
  nb = batch // BB
  assert nb * BB == batch and (BB * dim) % 128 == 0

  def trans(in_ref, out_ref):
    x = in_ref[0]
    parts = [x[:, q * dim:(q + 1) * dim].T for q in range(lanes_per_row)]
    out_ref[0] = jnp.concatenate(parts, axis=1)

  return pl.pallas_call(
      trans,
      grid=(hist, nb),
      in_specs=[pl.BlockSpec((1, packed_rows, 128), lambda h, j: (h, j, 0))],
      out_specs=pl.BlockSpec((1, dim, BB), lambda h, j: (h, 0, j)),
      out_shape=jax.ShapeDtypeStruct((hist, dim, batch), jnp.float32),
  )


def kernel(i, table):
  b, h = i.shape
  vocab, dim = table.shape
  n_total = b * h
  lanes_per_row = 128 // dim
  sub = BB // lanes_per_row
  idx_p = (i.T.reshape(h, b // BB, lanes_per_row, sub)
           .swapaxes(2, 3).reshape(n_total))
  tmp = _build_gather(n_total, vocab, dim)(idx_p, table)
  packed = tmp.reshape(h, b * dim // 128, 128)
  out_phys = _build_transpose(h, b, dim)(packed)
  return out_phys.transpose(2, 0, 1)
